# baseline (speedup 1.0000x reference)
import functools

import jax
import jax.numpy as jnp
from jax import lax
from jax.experimental import pallas as pl
from jax.experimental.pallas import tpu as pltpu
from jax.experimental.pallas import tpu_sc as plsc

_DIM = 2048
_E = 64
_K = 8
_G = 8
_EPG = _E // _G
_TG = 4
_SCALE = 2.5
_N = 8192
_BT = 256
_NBLK = _N // _BT
_NC = 2
_NS = 16
_NW = _NC * _NS
_FLAT = _N * _K
_CHUNK = _FLAT // _NW
_ROWS = _CHUNK // 16


def _tc_body(x_ref, gate_ref, bias_ref, sel_ref, pack_ref, pdest_ref,
             counts_ref, offs_ref, carry_ref):
    b = pl.program_id(0)

    @pl.when(b == 0)
    def _():
        carry_ref[...] = jnp.zeros_like(carry_ref)

    x = x_ref[...]
    gate = gate_ref[...]
    logits = lax.dot_general(x, gate, (((1,), (1,)), ((), ())),
                             preferred_element_type=jnp.float32)
    scores = jax.nn.sigmoid(logits)
    sfc = scores + bias_ref[...]

    lane = lax.broadcasted_iota(jnp.int32, (_BT, _E), 1)
    grp_of_lane = lane // _EPG
    neg = jnp.float32(-jnp.inf)

    t1 = sfc
    t2 = jnp.full((_BT, _E), neg)
    for k in (1, 2, 4):
        r1 = jnp.roll(t1, -k, axis=1)
        r2 = jnp.roll(t2, -k, axis=1)
        lo = jnp.minimum(t1, r1)
        t1 = jnp.maximum(t1, r1)
        t2 = jnp.maximum(jnp.maximum(t2, r2), lo)
    gsum = t1 + t2

    gidx = grp_of_lane
    rank = jnp.zeros((_BT, _E), jnp.float32)
    for k in range(1, _G):
        rg = jnp.roll(gsum, -_EPG * k, axis=1)
        wrapped = gidx >= _G - k
        beats = jnp.logical_or(rg > gsum,
                               jnp.logical_and(rg == gsum, wrapped))
        rank = rank + jnp.where(beats, 1.0, 0.0)
    gbit = jnp.where(jnp.logical_and(lane % _EPG == 0, rank < _TG), 1.0, 0.0)
    for k in (1, 2, 4):
        gbit = gbit + jnp.roll(gbit, k, axis=1)
    mask64 = gbit > 0.5

    masked = jnp.where(mask64, sfc, neg)
    sel_cols, onehots = [], []
    for _ in range(_K):
        m = jnp.max(masked, axis=1, keepdims=True)
        fi = jnp.min(jnp.where(masked == m, lane, _E), axis=1, keepdims=True)
        hit = lane == fi
        sel_cols.append(fi)
        onehots.append(hit)
        masked = jnp.where(hit, neg, masked)

    sel8 = jnp.concatenate(sel_cols, axis=1)

    r_bd = lax.broadcasted_iota(jnp.int32, (_K * _E, _K), 0)
    c_bd = lax.broadcasted_iota(jnp.int32, (_K * _E, _K), 1)
    bd = (r_bd // _E == c_bd).astype(jnp.float32)

    w_mat = jnp.concatenate(
        [jnp.where(onehots[s], scores, 0.0) for s in range(_K)], axis=1)
    w8 = lax.dot_general(w_mat, bd, (((1,), (0,)), ((), ())),
                         precision=lax.Precision.HIGHEST,
                         preferred_element_type=jnp.float32)
    denom = jnp.sum(w8, axis=1, keepdims=True) + 1e-20
    w8 = w8 / denom * _SCALE

    pack_ref[...] = w8

    counts = jnp.zeros((_BT, _E), jnp.float32)
    p_parts = []
    for s in range(_K):
        oh = onehots[s].astype(jnp.float32)
        p_parts.append(counts * oh)
        counts = counts + oh
    prior = lax.dot_general(jnp.concatenate(p_parts, axis=1), bd,
                            (((1,), (0,)), ((), ())),
                            precision=lax.Precision.HIGHEST,
                            preferred_element_type=jnp.float32)

    r_i = lax.broadcasted_iota(jnp.int32, (_BT, _BT), 0)
    c_i = lax.broadcasted_iota(jnp.int32, (_BT, _BT), 1)
    lstrict = (r_i > c_i).astype(jnp.float32)
    cexc = lax.dot_general(lstrict, counts, (((1,), (0,)), ((), ())),
                           precision=lax.Precision.HIGHEST,
                           preferred_element_type=jnp.float32)
    cexc = cexc + carry_ref[...]

    c_mat = jnp.concatenate(
        [jnp.where(onehots[s], cexc, 0.0) for s in range(_K)], axis=1)
    pdest = lax.dot_general(c_mat, bd, (((1,), (0,)), ((), ())),
                            precision=lax.Precision.HIGHEST,
                            preferred_element_type=jnp.float32) + prior

    sel_ref[...] = sel8
    pdest_ref[...] = pdest.astype(jnp.int32)

    new_carry = carry_ref[...] + jnp.sum(counts, axis=0, keepdims=True)
    carry_ref[...] = new_carry
    counts_ref[...] = new_carry.astype(jnp.int32)

    r_e = lax.broadcasted_iota(jnp.int32, (_E, _E), 0)
    c_e = lax.broadcasted_iota(jnp.int32, (_E, _E), 1)
    ustrict = (r_e < c_e).astype(jnp.float32)
    offs = lax.dot_general(new_carry, ustrict, (((1,), (0,)), ((), ())),
                           precision=lax.Precision.HIGHEST,
                           preferred_element_type=jnp.float32)
    offs_ref[...] = offs.astype(jnp.int32)


_tc_call = pl.pallas_call(
    _tc_body,
    grid=(_NBLK,),
    in_specs=[
        pl.BlockSpec((_BT, _DIM), lambda b: (b, 0)),
        pl.BlockSpec((_E, _DIM), lambda b: (0, 0)),
        pl.BlockSpec((1, _E), lambda b: (0, 0)),
    ],
    out_specs=[
        pl.BlockSpec((_BT, _K), lambda b: (b, 0)),
        pl.BlockSpec((_BT, _K), lambda b: (b, 0)),
        pl.BlockSpec((_BT, _K), lambda b: (b, 0)),
        pl.BlockSpec((1, _E), lambda b: (0, 0)),
        pl.BlockSpec((1, _E), lambda b: (0, 0)),
    ],
    out_shape=[
        jax.ShapeDtypeStruct((_N, _K), jnp.int32),
        jax.ShapeDtypeStruct((_N, _K), jnp.float32),
        jax.ShapeDtypeStruct((_N, _K), jnp.int32),
        jax.ShapeDtypeStruct((1, _E), jnp.int32),
        jax.ShapeDtypeStruct((1, _E), jnp.int32),
    ],
    scratch_shapes=[pltpu.VMEM((1, _E), jnp.float32)],
)


_SCCHUNK = _FLAT // _NS


def _sc_body(sel_hbm, pd_hbm, pay_hbm, off_hbm, spe_hbm, tis_hbm,
             image, shoffs, sel_v, pd_v, pay_v, offg_v, dest_v, sem):
    cid = lax.axis_index("c")
    sid = lax.axis_index("s")

    @pl.when(sid == 0)
    def _():
        pltpu.sync_copy(off_hbm, shoffs)

    pltpu.sync_copy(sel_hbm.at[sid], sel_v)
    pltpu.sync_copy(pd_hbm.at[sid], pd_v)
    pltpu.sync_copy(pay_hbm.at[cid, sid], pay_v)
    plsc.subcore_barrier()
    gat = pltpu.make_async_copy(shoffs.at[sel_v], offg_v, sem)
    gat.start()
    gat.wait()

    def body(j, carry):
        sl = pl.ds(j * 16, 16)
        dest_v[sl] = pd_v[sl] + offg_v[sl]
        return carry

    lax.fori_loop(0, _SCCHUNK // 16, body, 0)

    sc = pltpu.make_async_copy(pay_v, image.at[dest_v], sem)
    sc.start()
    sc.wait()
    plsc.subcore_barrier()
    sl = pl.ds(sid * _SCCHUNK, _SCCHUNK)

    @pl.when(cid == 0)
    def _():
        pltpu.sync_copy(image.at[sl], spe_hbm.at[sl])

    @pl.when(cid == 1)
    def _():
        pltpu.sync_copy(image.at[sl], tis_hbm.at[sl])


@functools.lru_cache(maxsize=1)
def _sc_dispatch():
    return pl.kernel(
        _sc_body,
        out_type=[
            jax.ShapeDtypeStruct((_FLAT,), jnp.int32),
            jax.ShapeDtypeStruct((_FLAT,), jnp.int32),
        ],
        mesh=plsc.VectorSubcoreMesh(core_axis_name="c", subcore_axis_name="s",
                                    num_cores=_NC, num_subcores=_NS),
        scratch_types=[
            pltpu.VMEM_SHARED((_FLAT,), jnp.int32),
            pltpu.VMEM_SHARED((_E,), jnp.int32),
            pltpu.VMEM((_SCCHUNK,), jnp.int32),
            pltpu.VMEM((_SCCHUNK,), jnp.int32),
            pltpu.VMEM((_SCCHUNK,), jnp.int32),
            pltpu.VMEM((_SCCHUNK,), jnp.int32),
            pltpu.VMEM((_SCCHUNK,), jnp.int32),
            pltpu.SemaphoreType.DMA,
        ],
    )


def kernel(x, gate, e_score_correction_bias):
    bias2 = e_score_correction_bias.reshape(1, _E)
    sel, w, pdest, counts, offs = _tc_call(x, gate, bias2)
    sel2 = sel.reshape(_NS, _SCCHUNK)
    pd2 = pdest.reshape(_NS, _SCCHUNK)
    wbits = lax.bitcast_convert_type(w, jnp.int32).reshape(_NS, _SCCHUNK)
    tok2 = (jnp.arange(_FLAT, dtype=jnp.int32) // _K).reshape(_NS, _SCCHUNK)
    pay = jnp.stack([wbits, tok2])
    spe_bits, tis = _sc_dispatch()(sel2, pd2, pay, offs.reshape(_E))
    spe = lax.bitcast_convert_type(spe_bits, jnp.float32)
    return spe, tis, counts.reshape(_E)


# speedup vs baseline: 1.3450x; 1.3450x over previous
import functools

import jax
import jax.numpy as jnp
from jax import lax
from jax.experimental import pallas as pl
from jax.experimental.pallas import tpu as pltpu
from jax.experimental.pallas import tpu_sc as plsc

_DIM = 2048
_E = 64
_K = 8
_G = 8
_EPG = _E // _G
_TG = 4
_SCALE = 2.5
_N = 8192
_BT = 256
_NBLK = _N // _BT
_NC = 2
_NS = 16
_NW = _NC * _NS
_FLAT = _N * _K
_CHUNK = _FLAT // _NW
_ROWS = _CHUNK // 16


def _tc_body(x_ref, gate_ref, bias_ref, sel_ref, pack_ref, pdest_ref,
             counts_ref, offs_ref, carry_ref):
    b = pl.program_id(0)

    @pl.when(b == 0)
    def _():
        carry_ref[...] = jnp.zeros_like(carry_ref)

    x = x_ref[...]
    gate = gate_ref[...]
    logits = lax.dot_general(x, gate, (((1,), (1,)), ((), ())),
                             preferred_element_type=jnp.float32)
    scores = jax.nn.sigmoid(logits)
    sfc = scores + bias_ref[...]

    lane = lax.broadcasted_iota(jnp.int32, (_BT, _E), 1)
    grp_of_lane = lane // _EPG
    neg = jnp.float32(-jnp.inf)

    t1 = sfc
    t2 = jnp.full((_BT, _E), neg)
    for k in (1, 2, 4):
        r1 = jnp.roll(t1, -k, axis=1)
        r2 = jnp.roll(t2, -k, axis=1)
        lo = jnp.minimum(t1, r1)
        t1 = jnp.maximum(t1, r1)
        t2 = jnp.maximum(jnp.maximum(t2, r2), lo)
    gsum = t1 + t2

    gidx = grp_of_lane
    rank = jnp.zeros((_BT, _E), jnp.float32)
    for k in range(1, _G):
        rg = jnp.roll(gsum, -_EPG * k, axis=1)
        wrapped = gidx >= _G - k
        beats = jnp.logical_or(rg > gsum,
                               jnp.logical_and(rg == gsum, wrapped))
        rank = rank + jnp.where(beats, 1.0, 0.0)
    gbit = jnp.where(jnp.logical_and(lane % _EPG == 0, rank < _TG), 1.0, 0.0)
    for k in (1, 2, 4):
        gbit = gbit + jnp.roll(gbit, k, axis=1)
    mask64 = gbit > 0.5

    masked = jnp.where(mask64, sfc, neg)
    sel_cols, sc_cols, onehots = [], [], []
    for _ in range(_K):
        m = jnp.max(masked, axis=1, keepdims=True)
        fi = jnp.min(jnp.where(masked == m, lane, _E), axis=1, keepdims=True)
        hit = lane == fi
        sel_cols.append(fi)
        sc_cols.append(jnp.sum(jnp.where(hit, scores, 0.0), axis=1,
                               keepdims=True))
        onehots.append(hit)
        masked = jnp.where(hit, neg, masked)

    sel8 = jnp.concatenate(sel_cols, axis=1)

    w8 = jnp.concatenate(sc_cols, axis=1)
    denom = jnp.sum(w8, axis=1, keepdims=True) + 1e-20
    w8 = w8 / denom * _SCALE

    pack_ref[...] = w8

    counts = jnp.zeros((_BT, _E), jnp.float32)
    prior_cols = []
    for s in range(_K):
        oh = onehots[s].astype(jnp.float32)
        prior_cols.append(jnp.sum(counts * oh, axis=1, keepdims=True))
        counts = counts + oh
    prior = jnp.concatenate(prior_cols, axis=1)

    r_i = lax.broadcasted_iota(jnp.int32, (_BT, _BT), 0)
    c_i = lax.broadcasted_iota(jnp.int32, (_BT, _BT), 1)
    lstrict = (r_i > c_i).astype(jnp.float32)
    cexc = lax.dot_general(lstrict, counts, (((1,), (0,)), ((), ())),
                           precision=lax.Precision.HIGHEST,
                           preferred_element_type=jnp.float32)
    cexc = cexc + carry_ref[...]

    pdest_cols = []
    for s in range(_K):
        oh = onehots[s].astype(jnp.float32)
        pdest_cols.append(jnp.sum(cexc * oh, axis=1, keepdims=True))
    pdest = jnp.concatenate(pdest_cols, axis=1) + prior

    sel_ref[...] = sel8
    pdest_ref[...] = pdest.astype(jnp.int32)

    new_carry = carry_ref[...] + jnp.sum(counts, axis=0, keepdims=True)
    carry_ref[...] = new_carry
    counts_ref[...] = new_carry.astype(jnp.int32)

    r_e = lax.broadcasted_iota(jnp.int32, (_E, _E), 0)
    c_e = lax.broadcasted_iota(jnp.int32, (_E, _E), 1)
    ustrict = (r_e < c_e).astype(jnp.float32)
    offs = lax.dot_general(new_carry, ustrict, (((1,), (0,)), ((), ())),
                           precision=lax.Precision.HIGHEST,
                           preferred_element_type=jnp.float32)
    offs_ref[...] = offs.astype(jnp.int32)


_tc_call = pl.pallas_call(
    _tc_body,
    grid=(_NBLK,),
    in_specs=[
        pl.BlockSpec((_BT, _DIM), lambda b: (b, 0)),
        pl.BlockSpec((_E, _DIM), lambda b: (0, 0)),
        pl.BlockSpec((1, _E), lambda b: (0, 0)),
    ],
    out_specs=[
        pl.BlockSpec((_BT, _K), lambda b: (b, 0)),
        pl.BlockSpec((_BT, _K), lambda b: (b, 0)),
        pl.BlockSpec((_BT, _K), lambda b: (b, 0)),
        pl.BlockSpec((1, _E), lambda b: (0, 0)),
        pl.BlockSpec((1, _E), lambda b: (0, 0)),
    ],
    out_shape=[
        jax.ShapeDtypeStruct((_N, _K), jnp.int32),
        jax.ShapeDtypeStruct((_N, _K), jnp.float32),
        jax.ShapeDtypeStruct((_N, _K), jnp.int32),
        jax.ShapeDtypeStruct((1, _E), jnp.int32),
        jax.ShapeDtypeStruct((1, _E), jnp.int32),
    ],
    scratch_shapes=[pltpu.VMEM((1, _E), jnp.float32)],
)


_SCCHUNK = _FLAT // _NS


def _sc_body(sel_hbm, pd_hbm, pay_hbm, off_hbm, spe_hbm, tis_hbm,
             image, shoffs, sel_v, pd_v, pay_v, offg_v, dest_v, sem):
    cid = lax.axis_index("c")
    sid = lax.axis_index("s")

    @pl.when(sid == 0)
    def _():
        pltpu.sync_copy(off_hbm, shoffs)

    pltpu.sync_copy(sel_hbm.at[sid], sel_v)
    pltpu.sync_copy(pd_hbm.at[sid], pd_v)
    pltpu.sync_copy(pay_hbm.at[cid, sid], pay_v)
    plsc.subcore_barrier()
    gat = pltpu.make_async_copy(shoffs.at[sel_v], offg_v, sem)
    gat.start()
    gat.wait()

    def body(j, carry):
        sl = pl.ds(j * 16, 16)
        dest_v[sl] = pd_v[sl] + offg_v[sl]
        return carry

    lax.fori_loop(0, _SCCHUNK // 16, body, 0)

    sc = pltpu.make_async_copy(pay_v, image.at[dest_v], sem)
    sc.start()
    sc.wait()
    plsc.subcore_barrier()
    sl = pl.ds(sid * _SCCHUNK, _SCCHUNK)

    @pl.when(cid == 0)
    def _():
        pltpu.sync_copy(image.at[sl], spe_hbm.at[sl])

    @pl.when(cid == 1)
    def _():
        pltpu.sync_copy(image.at[sl], tis_hbm.at[sl])


@functools.lru_cache(maxsize=1)
def _sc_dispatch():
    return pl.kernel(
        _sc_body,
        out_type=[
            jax.ShapeDtypeStruct((_FLAT,), jnp.int32),
            jax.ShapeDtypeStruct((_FLAT,), jnp.int32),
        ],
        mesh=plsc.VectorSubcoreMesh(core_axis_name="c", subcore_axis_name="s",
                                    num_cores=_NC, num_subcores=_NS),
        scratch_types=[
            pltpu.VMEM_SHARED((_FLAT,), jnp.int32),
            pltpu.VMEM_SHARED((_E,), jnp.int32),
            pltpu.VMEM((_SCCHUNK,), jnp.int32),
            pltpu.VMEM((_SCCHUNK,), jnp.int32),
            pltpu.VMEM((_SCCHUNK,), jnp.int32),
            pltpu.VMEM((_SCCHUNK,), jnp.int32),
            pltpu.VMEM((_SCCHUNK,), jnp.int32),
            pltpu.SemaphoreType.DMA,
        ],
    )


def kernel(x, gate, e_score_correction_bias):
    bias2 = e_score_correction_bias.reshape(1, _E)
    sel, w, pdest, counts, offs = _tc_call(x, gate, bias2)
    sel2 = sel.reshape(_NS, _SCCHUNK)
    pd2 = pdest.reshape(_NS, _SCCHUNK)
    wbits = lax.bitcast_convert_type(w, jnp.int32).reshape(_NS, _SCCHUNK)
    tok2 = (jnp.arange(_FLAT, dtype=jnp.int32) // _K).reshape(_NS, _SCCHUNK)
    pay = jnp.stack([wbits, tok2])
    spe_bits, tis = _sc_dispatch()(sel2, pd2, pay, offs.reshape(_E))
    spe = lax.bitcast_convert_type(spe_bits, jnp.float32)
    return spe, tis, counts.reshape(_E)
